# baseline (speedup 1.0000x reference)
import functools

import jax
import jax.numpy as jnp
from jax import lax
from jax.experimental import pallas as pl
from jax.experimental.pallas import tpu as pltpu
from jax.experimental.pallas import tpu_sc as plsc

T, D, F, E = 2048, 768, 3072, 8
BT = 256
NBLK = (2 * T) // BT + E
P = NBLK * BT

NC, NS = 2, 16
NW = NC * NS
TPW = T // NW


def _router_body(x_ref, wr_ref, br_ref, xb_ref, pos1_ref, pos2_ref,
                 pospk_ref, g1_ref, g2_ref, bexp_ref, valid_ref):
    x = x_ref[...]
    xb_ref[...] = x.astype(jnp.bfloat16)
    logits = lax.dot_general(x, wr_ref[...], (((1,), (1,)), ((), ())))
    logits = logits + br_ref[...]
    m = jnp.max(logits, axis=1, keepdims=True)
    p = jnp.exp(logits - m)
    probs = p / jnp.sum(p, axis=1, keepdims=True)

    idx = lax.broadcasted_iota(jnp.int32, (T, E), 1)
    m1 = jnp.max(probs, axis=1, keepdims=True)
    sel1 = jnp.min(jnp.where(probs >= m1, idx, E), axis=1, keepdims=True)
    oh1 = (idx == sel1)
    probs_rest = jnp.where(oh1, -1.0, probs)
    m2 = jnp.max(probs_rest, axis=1, keepdims=True)
    sel2 = jnp.min(jnp.where(probs_rest >= m2, idx, E), axis=1, keepdims=True)
    oh2 = (idx == sel2)

    lanes = jnp.ones((1, 16), jnp.float32)
    g1_ref[...] = (m1 / (m1 + m2)) * lanes
    g2_ref[...] = (m2 / (m1 + m2)) * lanes

    oh1f = oh1.astype(jnp.float32)
    oh2f = oh2.astype(jnp.float32)
    ohf = oh1f + oh2f

    cume = ohf
    shift = 1
    while shift < T:
        cume = cume + jnp.concatenate(
            [jnp.zeros((shift, E), jnp.float32), cume[:T - shift]], axis=0)
        shift *= 2
    counts = cume[T - 1:T, :]
    cume = cume - ohf

    nb = jnp.floor((counts + (BT - 1)) * (1.0 / BT))
    upper = (lax.broadcasted_iota(jnp.int32, (E, E), 0)
             <= lax.broadcasted_iota(jnp.int32, (E, E), 1)).astype(jnp.float32)
    cum_nb = lax.dot_general(nb, upper, (((1,), (0,)), ((), ())))
    row_off = (cum_nb - nb) * BT

    rank1 = jnp.sum(oh1f * cume, axis=1, keepdims=True)
    rank2 = jnp.sum(oh2f * cume, axis=1, keepdims=True)
    off1 = jnp.sum(oh1f * row_off, axis=1, keepdims=True)
    off2 = jnp.sum(oh2f * row_off, axis=1, keepdims=True)
    p1i = (off1 + rank1).astype(jnp.int32)
    p2i = (off2 + rank2).astype(jnp.int32)
    pos1_ref[...] = p1i
    pos2_ref[...] = p2i
    pospk_ref[...] = p1i + p2i * 8192

    bidx = lax.broadcasted_iota(jnp.int32, (NBLK, E), 0).astype(jnp.float32)
    bexp = jnp.sum((bidx >= cum_nb).astype(jnp.float32), axis=1, keepdims=True)
    bexp_ref[...] = jnp.minimum(bexp, E - 1).astype(jnp.int32)
    total_nb = cum_nb[:, E - 1:E]
    valid_ref[...] = (bidx[:, 0:1] < total_nb).astype(jnp.int32)


def _sc_comb_body(ys_hbm, pos1_hbm, pos2_hbm, g1_hbm, g2_hbm, out_hbm,
                  idx1_v, idx2_v, g1_v, g2_v, r1_v, r2_v, sem1, sem2):
    wid = lax.axis_index("s") * NC + lax.axis_index("c")
    base = wid * TPW
    pltpu.sync_copy(pos1_hbm.at[pl.ds(base, TPW)], idx1_v)
    cp1 = pltpu.async_copy(ys_hbm.at[idx1_v], r1_v, sem1)
    pltpu.sync_copy(pos2_hbm.at[pl.ds(base, TPW)], idx2_v)
    cp2 = pltpu.async_copy(ys_hbm.at[idx2_v], r2_v, sem2)
    pltpu.sync_copy(g1_hbm.at[pl.ds(base, TPW)], g1_v)
    pltpu.sync_copy(g2_hbm.at[pl.ds(base, TPW)], g2_v)
    cp1.wait()
    cp2.wait()

    @plsc.parallel_loop(0, TPW, 1, unroll=2)
    def _row(i):
        a1 = g1_v[i]
        a2 = g2_v[i]
        for j in range(D // 16):
            sl = pl.ds(j * 16, 16)
            r1_v[i, sl] = a1 * r1_v[i, sl] + a2 * r2_v[i, sl]
    pltpu.sync_copy(r1_v, out_hbm.at[pl.ds(base, TPW)])


def _ffn_body(be_ref, va_ref, xv_ref, pk_ref, w1_ref, w3_ref, w2_ref,
              gam_ref, bet_ref, ys_ref):
    b = pl.program_id(0)

    @pl.when(va_ref[b] != 0)
    def _():
        pk = pk_ref[...]
        p1 = lax.bitwise_and(pk, 8191)
        p2 = lax.shift_right_logical(pk, 13)
        slot = b * BT + lax.broadcasted_iota(jnp.int32, (1, BT), 1)
        m = ((p1 == slot) | (p2 == slot)).astype(jnp.bfloat16)
        xb = lax.dot_general(m, xv_ref[...], (((0,), (0,)), ((), ())),
                             preferred_element_type=jnp.float32)
        f2 = F // 2
        y = None
        for h in range(2):
            w1h = w1_ref[0, h * f2:(h + 1) * f2, :]
            w3h = w3_ref[0, h * f2:(h + 1) * f2, :]
            w2h = w2_ref[0, :, h * f2:(h + 1) * f2]
            h1 = lax.dot_general(xb, w1h, (((1,), (1,)), ((), ())))
            h3 = lax.dot_general(xb, w3h, (((1,), (1,)), ((), ())))
            a = h1 * jax.nn.sigmoid(h1) * h3
            yh = lax.dot_general(a, w2h, (((1,), (1,)), ((), ())))
            y = yh if y is None else y + yh
        mu = jnp.mean(y, axis=1, keepdims=True)
        yc = y - mu
        var = jnp.mean(yc * yc, axis=1, keepdims=True)
        ys_ref[...] = (yc * lax.rsqrt(var + 1e-5) * gam_ref[0]
                       + bet_ref[0])


def kernel(x, Wr, br, W1, W2, W3, gamma, beta):
    Bx, N, Dx = x.shape
    xf = x.reshape(T, D)

    xb, pos1, pos2, pospk, g1, g2, bexp, valid = pl.pallas_call(
        _router_body,
        out_shape=[
            jax.ShapeDtypeStruct((T, D), jnp.bfloat16),
            jax.ShapeDtypeStruct((T, 1), jnp.int32),
            jax.ShapeDtypeStruct((T, 1), jnp.int32),
            jax.ShapeDtypeStruct((T, 1), jnp.int32),
            jax.ShapeDtypeStruct((T, 16), jnp.float32),
            jax.ShapeDtypeStruct((T, 16), jnp.float32),
            jax.ShapeDtypeStruct((NBLK, 1), jnp.int32),
            jax.ShapeDtypeStruct((NBLK, 1), jnp.int32),
        ],
    )(xf, Wr, br.reshape(1, E))

    pos1f = pos1.reshape(T)
    pos2f = pos2.reshape(T)

    bexp_s = bexp.reshape(NBLK)
    valid_s = valid.reshape(NBLK)

    ys = pl.pallas_call(
        _ffn_body,
        grid_spec=pltpu.PrefetchScalarGridSpec(
            num_scalar_prefetch=2,
            grid=(NBLK,),
            in_specs=[
                pl.BlockSpec((T, D), lambda b, be, va: (0, 0)),
                pl.BlockSpec((T, 1), lambda b, be, va: (0, 0)),
                pl.BlockSpec((1, F, D), lambda b, be, va: (be[b], 0, 0)),
                pl.BlockSpec((1, F, D), lambda b, be, va: (be[b], 0, 0)),
                pl.BlockSpec((1, D, F), lambda b, be, va: (be[b], 0, 0)),
                pl.BlockSpec((1, 1, D), lambda b, be, va: (be[b], 0, 0)),
                pl.BlockSpec((1, 1, D), lambda b, be, va: (be[b], 0, 0)),
            ],
            out_specs=pl.BlockSpec(
                (BT, D),
                lambda b, be, va: (b * va[b] + (NBLK - 1) * (1 - va[b]), 0)),
        ),
        out_shape=jax.ShapeDtypeStruct((P, D), jnp.float32),
        compiler_params=pltpu.CompilerParams(
            vmem_limit_bytes=100 * 1024 * 1024),
    )(bexp_s, valid_s, xb, pospk, W1, W3, W2,
      gamma.reshape(E, 1, D), beta.reshape(E, 1, D))

    out = pl.kernel(
        _sc_comb_body,
        out_type=jax.ShapeDtypeStruct((T, D), jnp.float32),
        mesh=plsc.VectorSubcoreMesh(core_axis_name="c", subcore_axis_name="s"),
        scratch_types=[
            pltpu.VMEM((TPW,), jnp.int32),
            pltpu.VMEM((TPW,), jnp.int32),
            pltpu.VMEM((TPW, 16), jnp.float32),
            pltpu.VMEM((TPW, 16), jnp.float32),
            pltpu.VMEM((TPW, D), jnp.float32),
            pltpu.VMEM((TPW, D), jnp.float32),
            pltpu.SemaphoreType.DMA,
            pltpu.SemaphoreType.DMA,
        ],
    )(ys, pos1f, pos2f, g1, g2)

    return out.reshape(Bx, N, Dx)


# speedup vs baseline: 1.1567x; 1.1567x over previous
import jax
import jax.numpy as jnp
from jax import lax
from jax.experimental import pallas as pl
from jax.experimental.pallas import tpu as pltpu
from jax.experimental.pallas import tpu_sc as plsc

T, D, F, E = 2048, 768, 3072, 8
BT = 256
NBLK = (2 * T) // BT + E
P = NBLK * BT

NC, NS = 2, 16
NW = NC * NS
TPW = T // NW


def _router_body(x_ref, wr_ref, br_ref, pos1_ref, pos2_ref,
                 g1_ref, g2_ref, bexp_ref, valid_ref):
    x = x_ref[...]
    logits = lax.dot_general(x, wr_ref[...], (((1,), (1,)), ((), ())))
    logits = logits + br_ref[...]
    m = jnp.max(logits, axis=1, keepdims=True)
    p = jnp.exp(logits - m)
    probs = p / jnp.sum(p, axis=1, keepdims=True)

    idx = lax.broadcasted_iota(jnp.int32, (T, E), 1)
    m1 = jnp.max(probs, axis=1, keepdims=True)
    sel1 = jnp.min(jnp.where(probs >= m1, idx, E), axis=1, keepdims=True)
    oh1 = (idx == sel1)
    probs_rest = jnp.where(oh1, -1.0, probs)
    m2 = jnp.max(probs_rest, axis=1, keepdims=True)
    sel2 = jnp.min(jnp.where(probs_rest >= m2, idx, E), axis=1, keepdims=True)
    oh2 = (idx == sel2)

    lanes = jnp.ones((1, 16), jnp.float32)
    g1_ref[...] = (m1 / (m1 + m2)) * lanes
    g2_ref[...] = (m2 / (m1 + m2)) * lanes

    oh1f = oh1.astype(jnp.float32)
    oh2f = oh2.astype(jnp.float32)
    ohf = oh1f + oh2f

    cume = ohf
    shift = 1
    while shift < T:
        cume = cume + jnp.concatenate(
            [jnp.zeros((shift, E), jnp.float32), cume[:T - shift]], axis=0)
        shift *= 2
    counts = cume[T - 1:T, :]
    cume = cume - ohf

    nb = jnp.floor((counts + (BT - 1)) * (1.0 / BT))
    upper = (lax.broadcasted_iota(jnp.int32, (E, E), 0)
             <= lax.broadcasted_iota(jnp.int32, (E, E), 1)).astype(jnp.float32)
    cum_nb = lax.dot_general(nb, upper, (((1,), (0,)), ((), ())))
    row_off = (cum_nb - nb) * BT

    rank1 = jnp.sum(oh1f * cume, axis=1, keepdims=True)
    rank2 = jnp.sum(oh2f * cume, axis=1, keepdims=True)
    off1 = jnp.sum(oh1f * row_off, axis=1, keepdims=True)
    off2 = jnp.sum(oh2f * row_off, axis=1, keepdims=True)
    pos1_ref[...] = (off1 + rank1).astype(jnp.int32)
    pos2_ref[...] = (off2 + rank2).astype(jnp.int32)

    bidx = lax.broadcasted_iota(jnp.int32, (NBLK, E), 0).astype(jnp.float32)
    bexp = jnp.sum((bidx >= cum_nb).astype(jnp.float32), axis=1, keepdims=True)
    bexp_ref[...] = jnp.minimum(bexp, E - 1).astype(jnp.int32)
    total_nb = cum_nb[:, E - 1:E]
    valid_ref[...] = (bidx[:, 0:1] < total_nb).astype(jnp.int32)


def _sc_scatter_body(x_hbm, pos1_hbm, pos2_hbm, xs_hbm, idx_v, rows_v, sem):
    wid = lax.axis_index("s") * NC + lax.axis_index("c")
    base = wid * TPW
    pltpu.sync_copy(x_hbm.at[pl.ds(base, TPW)], rows_v)
    pltpu.sync_copy(pos1_hbm.at[pl.ds(base, TPW)], idx_v)
    pltpu.async_copy(rows_v, xs_hbm.at[idx_v], sem).wait()
    pltpu.sync_copy(pos2_hbm.at[pl.ds(base, TPW)], idx_v)
    pltpu.async_copy(rows_v, xs_hbm.at[idx_v], sem).wait()


def _sc_comb_body(ys_hbm, pos1_hbm, pos2_hbm, g1_hbm, g2_hbm, out_hbm,
                  idx1_v, idx2_v, g1_v, g2_v, r1_v, r2_v, sem1, sem2):
    wid = lax.axis_index("s") * NC + lax.axis_index("c")
    base = wid * TPW
    pltpu.sync_copy(pos1_hbm.at[pl.ds(base, TPW)], idx1_v)
    cp1 = pltpu.async_copy(ys_hbm.at[idx1_v], r1_v, sem1)
    pltpu.sync_copy(pos2_hbm.at[pl.ds(base, TPW)], idx2_v)
    cp2 = pltpu.async_copy(ys_hbm.at[idx2_v], r2_v, sem2)
    pltpu.sync_copy(g1_hbm.at[pl.ds(base, TPW)], g1_v)
    pltpu.sync_copy(g2_hbm.at[pl.ds(base, TPW)], g2_v)
    cp1.wait()
    cp2.wait()

    @plsc.parallel_loop(0, TPW, 1, unroll=2)
    def _row(i):
        a1 = g1_v[i]
        a2 = g2_v[i]
        for j in range(D // 16):
            sl = pl.ds(j * 16, 16)
            r1_v[i, sl] = a1 * r1_v[i, sl] + a2 * r2_v[i, sl]
    pltpu.sync_copy(r1_v, out_hbm.at[pl.ds(base, TPW)])


def _ffn_body(be_ref, va_ref, xs_ref, w1_ref, w3_ref, w2_ref,
              gam_ref, bet_ref, ys_ref):
    b = pl.program_id(0)

    @pl.when(va_ref[b] != 0)
    def _():
        xb = xs_ref[...]
        h1 = lax.dot_general(xb, w1_ref[0], (((1,), (1,)), ((), ())))
        h3 = lax.dot_general(xb, w3_ref[0], (((1,), (1,)), ((), ())))
        a = h1 * jax.nn.sigmoid(h1) * h3
        y = lax.dot_general(a, w2_ref[0], (((1,), (1,)), ((), ())))
        mu = jnp.mean(y, axis=1, keepdims=True)
        yc = y - mu
        var = jnp.mean(yc * yc, axis=1, keepdims=True)
        ys_ref[...] = (yc * lax.rsqrt(var + 1e-5) * gam_ref[0]
                       + bet_ref[0])


def kernel(x, Wr, br, W1, W2, W3, gamma, beta):
    Bx, N, Dx = x.shape
    xf = x.reshape(T, D)

    pos1, pos2, g1, g2, bexp, valid = pl.pallas_call(
        _router_body,
        out_shape=[
            jax.ShapeDtypeStruct((T, 1), jnp.int32),
            jax.ShapeDtypeStruct((T, 1), jnp.int32),
            jax.ShapeDtypeStruct((T, 16), jnp.float32),
            jax.ShapeDtypeStruct((T, 16), jnp.float32),
            jax.ShapeDtypeStruct((NBLK, 1), jnp.int32),
            jax.ShapeDtypeStruct((NBLK, 1), jnp.int32),
        ],
    )(xf, Wr, br.reshape(1, E))

    pos1f = pos1.reshape(T)
    pos2f = pos2.reshape(T)

    xs = pl.kernel(
        _sc_scatter_body,
        out_type=jax.ShapeDtypeStruct((P, D), jnp.float32),
        mesh=plsc.VectorSubcoreMesh(core_axis_name="c", subcore_axis_name="s"),
        scratch_types=[
            pltpu.VMEM((TPW,), jnp.int32),
            pltpu.VMEM((TPW, D), jnp.float32),
            pltpu.SemaphoreType.DMA,
        ],
    )(xf, pos1f, pos2f)

    bexp_s = bexp.reshape(NBLK)
    valid_s = valid.reshape(NBLK)

    ys = pl.pallas_call(
        _ffn_body,
        grid_spec=pltpu.PrefetchScalarGridSpec(
            num_scalar_prefetch=2,
            grid=(NBLK,),
            in_specs=[
                pl.BlockSpec((BT, D), lambda b, be, va: (b * va[b], 0)),
                pl.BlockSpec((1, F, D), lambda b, be, va: (be[b], 0, 0)),
                pl.BlockSpec((1, F, D), lambda b, be, va: (be[b], 0, 0)),
                pl.BlockSpec((1, D, F), lambda b, be, va: (be[b], 0, 0)),
                pl.BlockSpec((1, 1, D), lambda b, be, va: (be[b], 0, 0)),
                pl.BlockSpec((1, 1, D), lambda b, be, va: (be[b], 0, 0)),
            ],
            out_specs=pl.BlockSpec(
                (BT, D),
                lambda b, be, va: (b * va[b] + (NBLK - 1) * (1 - va[b]), 0)),
        ),
        out_shape=jax.ShapeDtypeStruct((P, D), jnp.float32),
        compiler_params=pltpu.CompilerParams(
            vmem_limit_bytes=100 * 1024 * 1024),
    )(bexp_s, valid_s, xs, W1, W3, W2,
      gamma.reshape(E, 1, D), beta.reshape(E, 1, D))

    out = pl.kernel(
        _sc_comb_body,
        out_type=jax.ShapeDtypeStruct((T, D), jnp.float32),
        mesh=plsc.VectorSubcoreMesh(core_axis_name="c", subcore_axis_name="s"),
        scratch_types=[
            pltpu.VMEM((TPW,), jnp.int32),
            pltpu.VMEM((TPW,), jnp.int32),
            pltpu.VMEM((TPW, 16), jnp.float32),
            pltpu.VMEM((TPW, 16), jnp.float32),
            pltpu.VMEM((TPW, D), jnp.float32),
            pltpu.VMEM((TPW, D), jnp.float32),
            pltpu.SemaphoreType.DMA,
            pltpu.SemaphoreType.DMA,
        ],
    )(ys, pos1f, pos2f, g1, g2)

    return out.reshape(Bx, N, Dx)
